# Initial kernel scaffold; baseline (speedup 1.0000x reference)
import jax
import jax.numpy as jnp
from jax.experimental import pallas as pl


def kernel(coords, features, phi_w, phi_b, psi_w, psi_b, g1_w, g1_b, g2_w, g2_b, s1_w, s1_b, s2_w, s2_b, a_w, a_b):
    raise NotImplementedError("write your pallas kernel here")


# speedup vs baseline: 10.7045x; 10.7045x over previous
import functools

import jax
import jax.numpy as jnp
from jax import lax
from jax.experimental import pallas as pl
from jax.experimental.pallas import tpu as pltpu
from jax.experimental.pallas import tpu_sc as plsc

_B, _N, _K, _C = 2, 4096, 16, 256
_TN = 256
_CP = 16


def _knn_body(ct_ref, cT_ref, idx_ref):
    b = pl.program_id(0)
    ct = ct_ref[0]
    cT = cT_ref[0]
    sq_t = jnp.sum(ct * ct, axis=1, keepdims=True)
    sq_all = jnp.sum(cT * cT, axis=0, keepdims=True)
    dots = lax.dot_general(ct, cT, (((1,), (0,)), ((), ())),
                           preferred_element_type=jnp.float32)
    d = sq_t + sq_all - 2.0 * dots
    iota = lax.broadcasted_iota(jnp.int32, (_TN, _N), 1)
    base = b * _N
    for k in range(_K):
        m = jnp.min(d, axis=1, keepdims=True)
        hit = d == m
        idx = jnp.min(jnp.where(hit, iota, _N), axis=1)
        idx_ref[0, k, :] = idx + base
        d = jnp.where(iota == idx[:, None], jnp.inf, d)


def _knn_topk(coords_pad, coords_T):
    grid = (_B, _N // _TN)
    return pl.pallas_call(
        _knn_body,
        grid=grid,
        in_specs=[
            pl.BlockSpec((1, _TN, _CP), lambda b, t: (b, t, 0)),
            pl.BlockSpec((1, _CP, _N), lambda b, t: (b, 0, 0)),
        ],
        out_specs=pl.BlockSpec((1, _K, _TN), lambda b, t: (b, 0, t)),
        out_shape=jax.ShapeDtypeStruct((_B, _K, _N), jnp.int32),
    )(coords_pad, coords_T)


_NC, _NS = 2, 16
_NW = _NC * _NS
_TOT = _B * _K * _N
_PER_W = _TOT // _NW


@functools.cache
def _sc_gather_kernel():
    @functools.partial(
        pl.kernel,
        mesh=plsc.VectorSubcoreMesh(core_axis_name="c", subcore_axis_name="s"),
        out_type=jax.ShapeDtypeStruct((_TOT, _CP), jnp.float32),
        compiler_params=pltpu.CompilerParams(use_tc_tiling_on_sc=False),
        scratch_types=[
            pltpu.VMEM((_PER_W,), jnp.int32),
            pltpu.VMEM((_PER_W, _CP), jnp.float32),
            pltpu.SemaphoreType.DMA,
        ],
    )
    def body(table_hbm, idx_hbm, out_hbm, idx_v, rows_v, sem):
        wid = lax.axis_index("s") * _NC + lax.axis_index("c")
        base = wid * _PER_W
        pltpu.sync_copy(idx_hbm.at[pl.ds(base, _PER_W)], idx_v)
        pltpu.async_copy(table_hbm.at[idx_v], rows_v, sem).wait()
        pltpu.sync_copy(rows_v, out_hbm.at[pl.ds(base, _PER_W)])

    return body


def _sc_gather(table, idx):
    return _sc_gather_kernel()(table, idx)


def _attn_body(f_ref, ci_ref, knn_ref, phiT_ref, g1T_ref, g2T_ref, acat_ref,
               vcat_ref, s1p_ref, s1b_ref, bcomb_ref, g1b_ref, g2b_ref,
               bv_ref, out_ref, af_s, uc_s, bu_s):
    b = pl.program_id(0)
    t = pl.program_id(1)

    @pl.when((b == 0) & (t == 0))
    def _():
        g1T = g1T_ref[...]
        af_s[...] = jnp.dot(phiT_ref[...], g1T,
                            preferred_element_type=jnp.float32)
        uc_s[...] = jnp.dot(acat_ref[...], g1T,
                            preferred_element_type=jnp.float32)
        bu_s[...] = jnp.dot(bcomb_ref[...], g1T,
                            preferred_element_type=jnp.float32) + g1b_ref[...]

    f = f_ref[0]
    fg = jnp.dot(f, af_s[...], preferred_element_type=jnp.float32) + bu_s[...]
    ci = ci_ref[0]
    g2T = g2T_ref[...]
    g2b = g2b_ref[...]
    bv = bv_ref[...]
    s1p = s1p_ref[...]
    s1b = s1b_ref[...]
    uc = uc_s[...]
    vc = vcat_ref[...]
    acc = jnp.zeros((_TN, _C), jnp.float32)
    for k in range(_K):
        cj = knn_ref[0, k]
        delta = ci - cj
        r1 = jnp.maximum(
            jnp.dot(delta, s1p, preferred_element_type=jnp.float32) + s1b, 0.0)
        cat = jnp.concatenate([cj, r1], axis=1)
        u = fg + jnp.dot(cat, uc, preferred_element_type=jnp.float32)
        h = jnp.maximum(u, 0.0)
        logit = jnp.dot(h, g2T, preferred_element_type=jnp.float32) + g2b
        mx = jnp.max(logit, axis=1, keepdims=True)
        e = jnp.exp(logit - mx)
        a = e / jnp.sum(e, axis=1, keepdims=True)
        v = jnp.dot(cat, vc, preferred_element_type=jnp.float32) + bv
        acc = acc + a * v
    out_ref[0] = acc


def _attn(features, coords_pad, knn_coords, phiT, g1T, g2T, a_cat, v_cat,
          s1p, s1b, b_comb, g1b, g2b, b_v):
    grid = (_B, _N // _TN)
    full = lambda shape: pl.BlockSpec(shape, lambda b, t: (0,) * len(shape))
    return pl.pallas_call(
        _attn_body,
        grid=grid,
        in_specs=[
            pl.BlockSpec((1, _TN, _C), lambda b, t: (b, t, 0)),
            pl.BlockSpec((1, _TN, _CP), lambda b, t: (b, t, 0)),
            pl.BlockSpec((1, _K, _TN, _CP), lambda b, t: (b, 0, t, 0)),
            full((_C, _C)), full((_C, _C)), full((_C, _C)),
            full((32, _C)), full((32, _C)),
            full((_CP, _CP)), full((1, _CP)),
            full((1, _C)), full((1, _C)), full((1, _C)), full((1, _C)),
        ],
        out_specs=pl.BlockSpec((1, _TN, _C), lambda b, t: (b, t, 0)),
        out_shape=jax.ShapeDtypeStruct((_B, _N, _C), jnp.float32),
        scratch_shapes=[
            pltpu.VMEM((_C, _C), jnp.float32),
            pltpu.VMEM((32, _C), jnp.float32),
            pltpu.VMEM((1, _C), jnp.float32),
        ],
    )(features, coords_pad, knn_coords, phiT, g1T, g2T, a_cat, v_cat,
      s1p, s1b, b_comb, g1b, g2b, b_v)


def kernel(coords, features, phi_w, phi_b, psi_w, psi_b, g1_w, g1_b,
           g2_w, g2_b, s1_w, s1_b, s2_w, s2_b, a_w, a_b):
    f32 = jnp.float32
    coords_pad = jnp.zeros((_B, _N, _CP), f32).at[:, :, :3].set(coords)
    coords_T = jnp.swapaxes(coords_pad, 1, 2)

    idx = _knn_topk(coords_pad, coords_T)
    table = coords_pad.reshape(_B * _N, _CP)
    knn_flat = _sc_gather(table, idx.reshape(_TOT))
    knn_coords = knn_flat.reshape(_B, _K, _N, _CP)

    pad3 = lambda w: jnp.zeros((_CP, _C), f32).at[:3, :].set(w.T)
    a_cat = jnp.concatenate([-pad3(psi_w), pad3(s2_w)], axis=0)
    v_cat = jnp.concatenate([pad3(a_w), pad3(s2_w)], axis=0)
    s1p = jnp.zeros((_CP, _CP), f32).at[:3, :3].set(s1_w.T)
    s1bp = jnp.zeros((1, _CP), f32).at[0, :3].set(s1_b)
    b_comb = (phi_b - psi_b + s2_b)[None, :]
    b_v = (a_b + s2_b)[None, :]

    return _attn(features, coords_pad, knn_coords,
                 phi_w.T, g1_w.T, g2_w.T, a_cat, v_cat,
                 s1p, s1bp, b_comb, g1b=g1_b[None, :], g2b=g2_b[None, :],
                 b_v=b_v)


# speedup vs baseline: 12.4441x; 1.1625x over previous
import functools

import jax
import jax.numpy as jnp
from jax import lax
from jax.experimental import pallas as pl
from jax.experimental.pallas import tpu as pltpu
from jax.experimental.pallas import tpu_sc as plsc

_B, _N, _K, _C = 2, 4096, 16, 256
_TN = 256
_CP = 16


def _knn_body(ct_ref, cT_ref, idx_ref):
    b = pl.program_id(0)
    ct = ct_ref[0]
    cT = cT_ref[0]
    sq_t = jnp.sum(ct * ct, axis=1, keepdims=True)
    sq_all = jnp.sum(cT * cT, axis=0, keepdims=True)
    dots = lax.dot_general(ct, cT, (((1,), (0,)), ((), ())),
                           preferred_element_type=jnp.float32)
    d = sq_t + sq_all - 2.0 * dots
    iota = lax.broadcasted_iota(jnp.int32, (_TN, _N), 1)
    t = pl.program_id(1)
    base = b * _N
    row = t * _TN + lax.broadcasted_iota(jnp.int32, (_TN,), 0)
    idx_ref[0, :, 0] = row + base
    d = jnp.where(iota == row[:, None], jnp.inf, d)
    for k in range(1, _K):
        idx = jnp.argmin(d, axis=1).astype(jnp.int32)
        idx_ref[0, :, k] = idx + base
        if k < _K - 1:
            d = jnp.where(iota == idx[:, None], jnp.inf, d)


def _knn_topk(coords_pad, coords_T):
    grid = (_B, _N // _TN)
    return pl.pallas_call(
        _knn_body,
        grid=grid,
        in_specs=[
            pl.BlockSpec((1, _TN, _CP), lambda b, t: (b, t, 0)),
            pl.BlockSpec((1, _CP, _N), lambda b, t: (b, 0, 0)),
        ],
        out_specs=pl.BlockSpec((1, _TN, _K), lambda b, t: (b, t, 0)),
        out_shape=jax.ShapeDtypeStruct((_B, _N, _K), jnp.int32),
    )(coords_pad, coords_T)


_NC, _NS = 2, 16
_NW = _NC * _NS
_TOT = _B * _K * _N
_PER_W = _TOT // _NW


@functools.cache
def _sc_gather_kernel():
    @functools.partial(
        pl.kernel,
        mesh=plsc.VectorSubcoreMesh(core_axis_name="c", subcore_axis_name="s"),
        out_type=jax.ShapeDtypeStruct((_TOT, _CP), jnp.float32),
        compiler_params=pltpu.CompilerParams(use_tc_tiling_on_sc=False),
        scratch_types=[
            pltpu.VMEM((_PER_W,), jnp.int32),
            pltpu.VMEM((_PER_W, _CP), jnp.float32),
            pltpu.SemaphoreType.DMA,
        ],
    )
    def body(table_hbm, idx_hbm, out_hbm, idx_v, rows_v, sem):
        wid = lax.axis_index("s") * _NC + lax.axis_index("c")
        base = wid * _PER_W
        pltpu.sync_copy(idx_hbm.at[pl.ds(base, _PER_W)], idx_v)
        pltpu.async_copy(table_hbm.at[idx_v], rows_v, sem).wait()
        pltpu.sync_copy(rows_v, out_hbm.at[pl.ds(base, _PER_W)])

    return body


def _sc_gather(table, idx):
    return _sc_gather_kernel()(table, idx)


def _attn_body(f_ref, ci_ref, knn_ref, phiT_ref, g1T_ref, g2T_ref, acat_ref,
               vcat_ref, s1p_ref, s1b_ref, bcomb_ref, g1b_ref, g2b_ref,
               bv_ref, out_ref, af_s, uc_s, bu_s):
    b = pl.program_id(0)
    t = pl.program_id(1)

    @pl.when((b == 0) & (t == 0))
    def _():
        g1T = g1T_ref[...]
        af_s[...] = jnp.dot(phiT_ref[...], g1T,
                            preferred_element_type=jnp.float32)
        uc_s[...] = jnp.dot(acat_ref[...], g1T,
                            preferred_element_type=jnp.float32)
        bu_s[...] = jnp.dot(bcomb_ref[...], g1T,
                            preferred_element_type=jnp.float32) + g1b_ref[...]

    f = f_ref[0]
    fg = jnp.dot(f, af_s[...], preferred_element_type=jnp.float32) + bu_s[...]
    ci = ci_ref[0]
    g2T = g2T_ref[...]
    g2b = g2b_ref[...]
    bv = bv_ref[...]
    s1p = s1p_ref[...]
    s1b = s1b_ref[...]
    uc = uc_s[...]
    vc = vcat_ref[...]
    acc = jnp.zeros((_TN, _C), jnp.float32)
    for k in range(_K):
        cj = knn_ref[0, :, k, :]
        delta = ci - cj
        r1 = jnp.maximum(
            jnp.dot(delta, s1p, preferred_element_type=jnp.float32) + s1b, 0.0)
        cat = jnp.concatenate([cj, r1], axis=1)
        u = fg + jnp.dot(cat, uc, preferred_element_type=jnp.float32)
        h = jnp.maximum(u, 0.0)
        logit = jnp.dot(h, g2T, preferred_element_type=jnp.float32) + g2b
        e = jnp.exp(logit)
        a = e * (1.0 / jnp.sum(e, axis=1, keepdims=True))
        v = jnp.dot(cat, vc, preferred_element_type=jnp.float32) + bv
        acc = acc + a * v
    out_ref[0] = acc


def _attn(features, coords_pad, knn_coords, phiT, g1T, g2T, a_cat, v_cat,
          s1p, s1b, b_comb, g1b, g2b, b_v):
    grid = (_B, _N // _TN)
    full = lambda shape: pl.BlockSpec(shape, lambda b, t: (0,) * len(shape))
    return pl.pallas_call(
        _attn_body,
        grid=grid,
        in_specs=[
            pl.BlockSpec((1, _TN, _C), lambda b, t: (b, t, 0)),
            pl.BlockSpec((1, _TN, _CP), lambda b, t: (b, t, 0)),
            pl.BlockSpec((1, _TN, _K, _CP), lambda b, t: (b, t, 0, 0)),
            full((_C, _C)), full((_C, _C)), full((_C, _C)),
            full((32, _C)), full((32, _C)),
            full((_CP, _CP)), full((1, _CP)),
            full((1, _C)), full((1, _C)), full((1, _C)), full((1, _C)),
        ],
        out_specs=pl.BlockSpec((1, _TN, _C), lambda b, t: (b, t, 0)),
        out_shape=jax.ShapeDtypeStruct((_B, _N, _C), jnp.float32),
        scratch_shapes=[
            pltpu.VMEM((_C, _C), jnp.float32),
            pltpu.VMEM((32, _C), jnp.float32),
            pltpu.VMEM((1, _C), jnp.float32),
        ],
    )(features, coords_pad, knn_coords, phiT, g1T, g2T, a_cat, v_cat,
      s1p, s1b, b_comb, g1b, g2b, b_v)


def kernel(coords, features, phi_w, phi_b, psi_w, psi_b, g1_w, g1_b,
           g2_w, g2_b, s1_w, s1_b, s2_w, s2_b, a_w, a_b):
    f32 = jnp.float32
    coords_pad = jnp.zeros((_B, _N, _CP), f32).at[:, :, :3].set(coords)
    coords_T = jnp.swapaxes(coords_pad, 1, 2)

    idx = _knn_topk(coords_pad, coords_T)
    table = coords_pad.reshape(_B * _N, _CP)
    knn_flat = _sc_gather(table, idx.reshape(_TOT))
    knn_coords = knn_flat.reshape(_B, _N, _K, _CP)

    pad3 = lambda w: jnp.zeros((_CP, _C), f32).at[:3, :].set(w.T)
    a_cat = jnp.concatenate([-pad3(psi_w), pad3(s2_w)], axis=0)
    v_cat = jnp.concatenate([pad3(a_w), pad3(s2_w)], axis=0)
    s1p = jnp.zeros((_CP, _CP), f32).at[:3, :3].set(s1_w.T)
    s1bp = jnp.zeros((1, _CP), f32).at[0, :3].set(s1_b)
    b_comb = (phi_b - psi_b + s2_b)[None, :]
    b_v = (a_b + s2_b)[None, :]

    return _attn(features, coords_pad, knn_coords,
                 phi_w.T, g1_w.T, g2_w.T, a_cat, v_cat,
                 s1p, s1bp, b_comb, g1b=g1_b[None, :], g2b=g2_b[None, :],
                 b_v=b_v)


# speedup vs baseline: 13.4937x; 1.0843x over previous
import functools

import jax
import jax.numpy as jnp
from jax import lax
from jax.experimental import pallas as pl
from jax.experimental.pallas import tpu as pltpu
from jax.experimental.pallas import tpu_sc as plsc

_B, _N, _K, _C = 2, 4096, 16, 256
_TN = 256
_CP = 16


def _knn_body(ct_ref, cT_ref, idx_ref):
    b = pl.program_id(0)
    ct = ct_ref[0]
    cT = cT_ref[0]
    sq_t = jnp.sum(ct * ct, axis=1, keepdims=True)
    sq_all = jnp.sum(cT * cT, axis=0, keepdims=True)
    dots = lax.dot_general(ct, cT, (((1,), (0,)), ((), ())),
                           preferred_element_type=jnp.float32)
    d = sq_t + sq_all - 2.0 * dots
    t16 = jnp.min(d[:, 0:_N // 16], axis=1, keepdims=True)
    for i in range(1, 16):
        blk = d[:, i * (_N // 16):(i + 1) * (_N // 16)]
        t16 = jnp.maximum(t16, jnp.min(blk, axis=1, keepdims=True))
    scale = 524287.0 / jnp.maximum(t16, 1e-30)
    q = jnp.clip(d * scale, 0.0, 524287.0).astype(jnp.int32)
    iota = lax.broadcasted_iota(jnp.int32, (_TN, _N), 1)
    t = pl.program_id(1)
    base = b * _N
    row = t * _TN + lax.broadcasted_iota(jnp.int32, (_TN,), 0)
    big = jnp.int32(2**31 - 1)
    key = (q << 12) | iota
    idx_ref[0, :, 0] = row + base
    key = jnp.where(iota == row[:, None], big, key)
    for k in range(1, _K):
        kmin = jnp.min(key, axis=1)
        idx_ref[0, :, k] = (kmin & 4095) + base
        if k < _K - 1:
            key = jnp.where(key == kmin[:, None], big, key)


def _knn_topk(coords_pad, coords_T):
    grid = (_B, _N // _TN)
    return pl.pallas_call(
        _knn_body,
        grid=grid,
        in_specs=[
            pl.BlockSpec((1, _TN, _CP), lambda b, t: (b, t, 0)),
            pl.BlockSpec((1, _CP, _N), lambda b, t: (b, 0, 0)),
        ],
        out_specs=pl.BlockSpec((1, _TN, _K), lambda b, t: (b, t, 0)),
        out_shape=jax.ShapeDtypeStruct((_B, _N, _K), jnp.int32),
    )(coords_pad, coords_T)


_NC, _NS = 2, 16
_NW = _NC * _NS
_TOT = _B * _K * _N
_PER_W = _TOT // _NW


@functools.cache
def _sc_gather_kernel():
    @functools.partial(
        pl.kernel,
        mesh=plsc.VectorSubcoreMesh(core_axis_name="c", subcore_axis_name="s"),
        out_type=jax.ShapeDtypeStruct((_TOT, _CP), jnp.float32),
        compiler_params=pltpu.CompilerParams(use_tc_tiling_on_sc=False),
        scratch_types=[
            pltpu.VMEM((_PER_W,), jnp.int32),
            pltpu.VMEM((_PER_W, _CP), jnp.float32),
            pltpu.SemaphoreType.DMA,
        ],
    )
    def body(table_hbm, idx_hbm, out_hbm, idx_v, rows_v, sem):
        wid = lax.axis_index("s") * _NC + lax.axis_index("c")
        base = wid * _PER_W
        pltpu.sync_copy(idx_hbm.at[pl.ds(base, _PER_W)], idx_v)
        pltpu.async_copy(table_hbm.at[idx_v], rows_v, sem).wait()
        pltpu.sync_copy(rows_v, out_hbm.at[pl.ds(base, _PER_W)])

    return body


def _sc_gather(table, idx):
    return _sc_gather_kernel()(table, idx)


def _attn_body(f_ref, ci_ref, knn_ref, phiT_ref, g1T_ref, g2bf_ref, acat_ref,
               vcat_ref, s1p_ref, s1b_ref, bcomb_ref, g1b_ref, g2b_ref,
               bv_ref, out_ref, af_s, uc_s, bu_s):
    b = pl.program_id(0)
    t = pl.program_id(1)

    @pl.when((b == 0) & (t == 0))
    def _():
        g1T = g1T_ref[...]
        af_s[...] = jnp.dot(phiT_ref[...], g1T,
                            preferred_element_type=jnp.float32)
        uc_s[...] = jnp.dot(acat_ref[...], g1T,
                            preferred_element_type=jnp.float32)
        bu_s[...] = jnp.dot(bcomb_ref[...], g1T,
                            preferred_element_type=jnp.float32) + g1b_ref[...]

    f = f_ref[0]
    fg = jnp.dot(f, af_s[...], preferred_element_type=jnp.float32) + bu_s[...]
    ci = ci_ref[0]
    g2bf = g2bf_ref[...]
    g2b = g2b_ref[...]
    bv = bv_ref[...]
    s1p = s1p_ref[...]
    s1b = s1b_ref[...]
    uc = uc_s[...]
    vc = vcat_ref[...]
    acc = jnp.zeros((_TN, _C), jnp.float32)
    for k in range(_K):
        cj = knn_ref[0, :, k, :]
        delta = ci - cj
        r1 = jnp.maximum(
            jnp.dot(delta, s1p, preferred_element_type=jnp.float32) + s1b, 0.0)
        cat = jnp.concatenate([cj, r1], axis=1)
        u = fg + jnp.dot(cat, uc, preferred_element_type=jnp.float32)
        h = jnp.maximum(u, 0.0).astype(jnp.bfloat16)
        logit = jnp.dot(h, g2bf, preferred_element_type=jnp.float32) + g2b
        e = jnp.exp(logit)
        a = e * (1.0 / jnp.sum(e, axis=1, keepdims=True))
        v = jnp.dot(cat, vc, preferred_element_type=jnp.float32) + bv
        acc = acc + a * v
    out_ref[0] = acc


def _attn(features, coords_pad, knn_coords, phiT, g1T, g2T, a_cat, v_cat,
          s1p, s1b, b_comb, g1b, g2b, b_v):
    grid = (_B, _N // _TN)
    full = lambda shape: pl.BlockSpec(shape, lambda b, t: (0,) * len(shape))
    return pl.pallas_call(
        _attn_body,
        grid=grid,
        in_specs=[
            pl.BlockSpec((1, _TN, _C), lambda b, t: (b, t, 0)),
            pl.BlockSpec((1, _TN, _CP), lambda b, t: (b, t, 0)),
            pl.BlockSpec((1, _TN, _K, _CP), lambda b, t: (b, t, 0, 0)),
            full((_C, _C)), full((_C, _C)), full((_C, _C)),
            full((32, _C)), full((32, _C)),
            full((_CP, _CP)), full((1, _CP)),
            full((1, _C)), full((1, _C)), full((1, _C)), full((1, _C)),
        ],
        out_specs=pl.BlockSpec((1, _TN, _C), lambda b, t: (b, t, 0)),
        out_shape=jax.ShapeDtypeStruct((_B, _N, _C), jnp.float32),
        scratch_shapes=[
            pltpu.VMEM((_C, _C), jnp.float32),
            pltpu.VMEM((32, _C), jnp.float32),
            pltpu.VMEM((1, _C), jnp.float32),
        ],
    )(features, coords_pad, knn_coords, phiT, g1T, g2T, a_cat, v_cat,
      s1p, s1b, b_comb, g1b, g2b, b_v)


def kernel(coords, features, phi_w, phi_b, psi_w, psi_b, g1_w, g1_b,
           g2_w, g2_b, s1_w, s1_b, s2_w, s2_b, a_w, a_b):
    f32 = jnp.float32
    coords_pad = jnp.zeros((_B, _N, _CP), f32).at[:, :, :3].set(coords)
    coords_T = jnp.swapaxes(coords_pad, 1, 2)

    idx = _knn_topk(coords_pad, coords_T)
    table = coords_pad.reshape(_B * _N, _CP)
    knn_flat = _sc_gather(table, idx.reshape(_TOT))
    knn_coords = knn_flat.reshape(_B, _N, _K, _CP)

    pad3 = lambda w: jnp.zeros((_CP, _C), f32).at[:3, :].set(w.T)
    a_cat = jnp.concatenate([-pad3(psi_w), pad3(s2_w)], axis=0)
    v_cat = jnp.concatenate([pad3(a_w), pad3(s2_w)], axis=0)
    s1p = jnp.zeros((_CP, _CP), f32).at[:3, :3].set(s1_w.T)
    s1bp = jnp.zeros((1, _CP), f32).at[0, :3].set(s1_b)
    b_comb = (phi_b - psi_b + s2_b)[None, :]
    b_v = (a_b + s2_b)[None, :]

    return _attn(features, coords_pad, knn_coords,
                 phi_w.T, g1_w.T, g2_w.T.astype(jnp.bfloat16), a_cat, v_cat,
                 s1p, s1bp, b_comb, g1b=g1_b[None, :], g2b=g2_b[None, :],
                 b_v=b_v)


# speedup vs baseline: 17.2761x; 1.2803x over previous
import functools

import jax
import jax.numpy as jnp
from jax import lax
from jax.experimental import pallas as pl
from jax.experimental.pallas import tpu as pltpu
from jax.experimental.pallas import tpu_sc as plsc

_B, _N, _K, _C = 2, 4096, 16, 256
_TN = 256
_CP = 16


def _knn_body(ct_ref, cT_ref, idx_ref):
    b = pl.program_id(0)
    ct = ct_ref[0]
    cT = cT_ref[0]
    sq_t = jnp.sum(ct * ct, axis=1, keepdims=True)
    sq_all = jnp.sum(cT * cT, axis=0, keepdims=True)
    dots = lax.dot_general(ct, cT, (((1,), (0,)), ((), ())),
                           preferred_element_type=jnp.float32)
    d = sq_t + sq_all - 2.0 * dots
    t16 = jnp.min(d[:, 0:_N // 16], axis=1, keepdims=True)
    for i in range(1, 16):
        blk = d[:, i * (_N // 16):(i + 1) * (_N // 16)]
        t16 = jnp.maximum(t16, jnp.min(blk, axis=1, keepdims=True))
    scale = 524287.0 / jnp.maximum(t16, 1e-30)
    q = jnp.clip(d * scale, 0.0, 524287.0).astype(jnp.int32)
    iota = lax.broadcasted_iota(jnp.int32, (_TN, _N), 1)
    t = pl.program_id(1)
    base = b * _N
    row = t * _TN + lax.broadcasted_iota(jnp.int32, (_TN,), 0)
    big = jnp.int32(2**31 - 1)
    key = (q << 12) | iota
    idx_ref[0, :, 0] = row + base
    key = jnp.where(iota == row[:, None], big, key)
    for k in range(1, _K):
        kmin = jnp.min(key, axis=1)
        idx_ref[0, :, k] = (kmin & 4095) + base
        if k < _K - 1:
            key = jnp.where(key == kmin[:, None], big, key)


def _knn_topk(coords_pad, coords_T):
    grid = (_B, _N // _TN)
    return pl.pallas_call(
        _knn_body,
        grid=grid,
        in_specs=[
            pl.BlockSpec((1, _TN, _CP), lambda b, t: (b, t, 0)),
            pl.BlockSpec((1, _CP, _N), lambda b, t: (b, 0, 0)),
        ],
        out_specs=pl.BlockSpec((1, _TN, _K), lambda b, t: (b, t, 0)),
        out_shape=jax.ShapeDtypeStruct((_B, _N, _K), jnp.int32),
    )(coords_pad, coords_T)


_NC, _NS = 2, 16
_NW = _NC * _NS
_TOT = _B * _K * _N
_PER_W = _TOT // _NW


@functools.cache
def _sc_gather_kernel():
    @functools.partial(
        pl.kernel,
        mesh=plsc.VectorSubcoreMesh(core_axis_name="c", subcore_axis_name="s"),
        out_type=jax.ShapeDtypeStruct((_TOT, _CP), jnp.float32),
        compiler_params=pltpu.CompilerParams(use_tc_tiling_on_sc=False),
        scratch_types=[
            pltpu.VMEM((_PER_W,), jnp.int32),
            pltpu.VMEM((_PER_W, _CP), jnp.float32),
            pltpu.SemaphoreType.DMA,
        ],
    )
    def body(table_hbm, idx_hbm, out_hbm, idx_v, rows_v, sem):
        wid = lax.axis_index("s") * _NC + lax.axis_index("c")
        base = wid * _PER_W
        pltpu.sync_copy(idx_hbm.at[pl.ds(base, _PER_W)], idx_v)
        pltpu.async_copy(table_hbm.at[idx_v], rows_v, sem).wait()
        pltpu.sync_copy(rows_v, out_hbm.at[pl.ds(base, _PER_W)])

    return body


def _sc_gather(table, idx):
    return _sc_gather_kernel()(table, idx)


def _attn_body(f_ref, ci_ref, knn_ref, phiT_ref, g1T_ref, g2bf_ref, acat_ref,
               vcat_ref, s1p_ref, s1b_ref, bcomb_ref, g1b_ref, g2b_ref,
               bv_ref, out_ref, af_s, uc_s, bu_s):
    b = pl.program_id(0)
    t = pl.program_id(1)

    @pl.when((b == 0) & (t == 0))
    def _():
        g1T = g1T_ref[...]
        af_s[...] = jnp.dot(phiT_ref[...], g1T,
                            preferred_element_type=jnp.float32)
        uc_s[...] = jnp.dot(acat_ref[...], g1T,
                            preferred_element_type=jnp.float32)
        bu_s[...] = jnp.dot(bcomb_ref[...], g1T,
                            preferred_element_type=jnp.float32) + g1b_ref[...]

    f = f_ref[0]
    fg = jnp.dot(f, af_s[...], preferred_element_type=jnp.float32) + bu_s[...]
    ci = ci_ref[0]
    g2bf = g2bf_ref[...]
    g2b = g2b_ref[...]
    bv = bv_ref[...]
    s1p = s1p_ref[...]
    s1b = s1b_ref[...]
    uc = uc_s[...]
    vc = vcat_ref[...]
    knn_slab = knn_ref[0].reshape(_K * _TN, _CP)
    cit = jnp.concatenate([ci] * _K, axis=0)
    delta = cit - knn_slab
    r1 = jnp.maximum(
        jnp.dot(delta, s1p, preferred_element_type=jnp.float32) + s1b, 0.0)
    cat = jnp.concatenate([knn_slab, r1], axis=1)
    uvu = jnp.dot(cat, uc, preferred_element_type=jnp.float32)
    uvv = jnp.dot(cat, vc, preferred_element_type=jnp.float32) + bv
    hs = []
    for k in range(_K):
        u = fg + uvu[k * _TN:(k + 1) * _TN]
        hs.append(jnp.maximum(u, 0.0).astype(jnp.bfloat16))
    h_slab = jnp.concatenate(hs, axis=0)
    logit = jnp.dot(h_slab, g2bf, preferred_element_type=jnp.float32) + g2b
    e = jnp.exp(logit)
    r = 1.0 / jnp.sum(e, axis=1, keepdims=True)
    acc = jnp.zeros((_TN, _C), jnp.float32)
    for k in range(_K):
        sl = slice(k * _TN, (k + 1) * _TN)
        acc = acc + (e[sl] * r[sl]) * uvv[sl]
    out_ref[0] = acc


def _attn(features, coords_pad, knn_coords, phiT, g1T, g2T, a_cat, v_cat,
          s1p, s1b, b_comb, g1b, g2b, b_v):
    grid = (_B, _N // _TN)
    full = lambda shape: pl.BlockSpec(shape, lambda b, t: (0,) * len(shape))
    return pl.pallas_call(
        _attn_body,
        grid=grid,
        in_specs=[
            pl.BlockSpec((1, _TN, _C), lambda b, t: (b, t, 0)),
            pl.BlockSpec((1, _TN, _CP), lambda b, t: (b, t, 0)),
            pl.BlockSpec((1, _K, _TN, _CP), lambda b, t: (b, 0, t, 0)),
            full((_C, _C)), full((_C, _C)), full((_C, _C)),
            full((32, _C)), full((32, _C)),
            full((_CP, _CP)), full((1, _CP)),
            full((1, _C)), full((1, _C)), full((1, _C)), full((1, _C)),
        ],
        out_specs=pl.BlockSpec((1, _TN, _C), lambda b, t: (b, t, 0)),
        out_shape=jax.ShapeDtypeStruct((_B, _N, _C), jnp.float32),
        scratch_shapes=[
            pltpu.VMEM((_C, _C), jnp.float32),
            pltpu.VMEM((32, _C), jnp.float32),
            pltpu.VMEM((1, _C), jnp.float32),
        ],
    )(features, coords_pad, knn_coords, phiT, g1T, g2T, a_cat, v_cat,
      s1p, s1b, b_comb, g1b, g2b, b_v)


def kernel(coords, features, phi_w, phi_b, psi_w, psi_b, g1_w, g1_b,
           g2_w, g2_b, s1_w, s1_b, s2_w, s2_b, a_w, a_b):
    f32 = jnp.float32
    coords_pad = jnp.zeros((_B, _N, _CP), f32).at[:, :, :3].set(coords)
    coords_T = jnp.swapaxes(coords_pad, 1, 2)

    idx = _knn_topk(coords_pad, coords_T)
    idx_km = jnp.swapaxes(idx, 1, 2)
    table = coords_pad.reshape(_B * _N, _CP)
    knn_flat = _sc_gather(table, idx_km.reshape(_TOT))
    knn_coords = knn_flat.reshape(_B, _K, _N, _CP)

    pad3 = lambda w: jnp.zeros((_CP, _C), f32).at[:3, :].set(w.T)
    a_cat = jnp.concatenate([-pad3(psi_w), pad3(s2_w)], axis=0)
    v_cat = jnp.concatenate([pad3(a_w), pad3(s2_w)], axis=0)
    s1p = jnp.zeros((_CP, _CP), f32).at[:3, :3].set(s1_w.T)
    s1bp = jnp.zeros((1, _CP), f32).at[0, :3].set(s1_b)
    b_comb = (phi_b - psi_b + s2_b)[None, :]
    b_v = (a_b + s2_b)[None, :]

    return _attn(features, coords_pad, knn_coords,
                 phi_w.T, g1_w.T, g2_w.T.astype(jnp.bfloat16), a_cat, v_cat,
                 s1p, s1bp, b_comb, g1b=g1_b[None, :], g2b=g2_b[None, :],
                 b_v=b_v)
